# Initial kernel scaffold; baseline (speedup 1.0000x reference)
import jax
import jax.numpy as jnp
from jax.experimental import pallas as pl


def kernel(x, edge_index, batch, target, c1_rel_w, c1_rel_b, c1_root_w, p1_w, c2_rel_w, c2_rel_b, c2_root_w, p2_w, cv_w, cv_b, gru_wih_f, gru_whh_f, gru_bih_f, gru_bhh_f, gru_wih_r, gru_whh_r, gru_bih_r, gru_bhh_r, d1_w, d1_b, d3_w, d3_b):
    raise NotImplementedError("write your pallas kernel here")


# speedup vs baseline: 1.3696x; 1.3696x over previous
import functools

import jax
import jax.numpy as jnp
from jax import lax
from jax.experimental import pallas as pl
from jax.experimental.pallas import tpu as pltpu
from jax.experimental.pallas import tpu_sc as plsc

B = 50; N0 = 10000; NPG = 200; K1 = 160; K2 = 128; E = 160000

NPAD = 10240
NOWN = 32
OWNR = NPAD // NOWN
HALF = NPAD // 2
CAP = 6144
CL = 128
NCHUNK = CAP // CL
PADROW = HALF


def _make_edge_scatter(F):
    mesh = plsc.VectorSubcoreMesh(core_axis_name="c", subcore_axis_name="s")

    @functools.partial(
        pl.kernel, mesh=mesh,
        out_type=jax.ShapeDtypeStruct((NPAD, F), jnp.float32),
        scratch_types=[
            pltpu.VMEM((NCHUNK, CL), jnp.int32),
            pltpu.VMEM((NCHUNK, CL), jnp.int32),
            pltpu.VMEM((CL, F), jnp.float32),
            pltpu.VMEM_SHARED((HALF + 8, F), jnp.float32),
            pltpu.SemaphoreType.DMA,
        ],
    )
    def k(src_hbm, dst_hbm, rows_hbm, zeros_hbm, out_hbm,
          src_v, dst_v, rows_v, agg_sh, sem):
        c = lax.axis_index("c")
        s = lax.axis_index("s")
        base = s * OWNR
        for z in range(OWNR // 64):
            pltpu.sync_copy(zeros_hbm, agg_sh.at[pl.ds(base + z * 64, 64)])
        pltpu.sync_copy(src_hbm.at[c, s], src_v)
        pltpu.sync_copy(dst_hbm.at[c, s], dst_v)
        plsc.subcore_barrier()

        def chunk(j, carry):
            pltpu.async_copy(rows_hbm.at[src_v.at[j]], rows_v, sem).wait()
            pltpu.sync_copy(rows_v, agg_sh.at[dst_v.at[j]], add=True)
            return carry

        lax.fori_loop(0, NCHUNK, chunk, 0)
        plsc.subcore_barrier()
        pltpu.sync_copy(agg_sh.at[pl.ds(base, OWNR)],
                        out_hbm.at[pl.ds(c * HALF + base, OWNR)])

    return k


_scatter128 = _make_edge_scatter(128)


def _bin_edges(src, dst):
    src = src.astype(jnp.int32)
    dst = dst.astype(jnp.int32)
    owner = dst // OWNR
    order = jnp.argsort(owner, stable=True).astype(jnp.int32)
    src_s = src[order]
    dst_s = dst[order]
    owner_s = owner[order]
    counts = jnp.bincount(owner, length=NOWN)
    starts = jnp.concatenate([jnp.zeros((1,), counts.dtype),
                              jnp.cumsum(counts)[:-1]])
    slot = jnp.arange(E, dtype=jnp.int32) - starts[owner_s].astype(jnp.int32)
    flat = owner_s * CAP + slot
    src_p = jnp.full((NOWN * CAP,), N0, jnp.int32).at[flat].set(
        src_s, unique_indices=True)
    dst_l = dst_s - (owner_s // 16) * HALF
    dst_p = jnp.full((NOWN * CAP,), PADROW, jnp.int32).at[flat].set(
        dst_l, unique_indices=True)
    shape = (2, 16, NCHUNK, CL)
    return src_p.reshape(shape), dst_p.reshape(shape)


def _gru_dir(xseq, Wih, Whh, bih, bhh, reverse):
    xs = jnp.transpose(xseq, (1, 0, 2))
    if reverse:
        xs = xs[::-1]
    gi = xs @ Wih.T + bih
    h0 = jnp.zeros((xseq.shape[0], 120), xseq.dtype)

    def step(h, g):
        gh = h @ Whh.T + bhh
        ir, iz, inn = jnp.split(g, 3, axis=-1)
        hr, hz, hn = jnp.split(gh, 3, axis=-1)
        r = jax.nn.sigmoid(ir + hr)
        z = jax.nn.sigmoid(iz + hz)
        n = jnp.tanh(inn + r * hn)
        hnew = (1 - z) * n + z * h
        return hnew, hnew

    _, hs = lax.scan(step, h0, gi)
    if reverse:
        hs = hs[::-1]
    return jnp.transpose(hs, (1, 0, 2))


def kernel(x, edge_index, batch, target, c1_rel_w, c1_rel_b, c1_root_w, p1_w,
           c2_rel_w, c2_rel_b, c2_root_w, p2_w, cv_w, cv_b,
           gru_wih_f, gru_whh_f, gru_bih_f, gru_bhh_f,
           gru_wih_r, gru_whh_r, gru_bih_r, gru_bhh_r, d1_w, d1_b, d3_w, d3_b):
    src, dst = edge_index[0], edge_index[1]
    src_g, dst_l = _bin_edges(src, dst)
    z = jnp.zeros((64, 128), jnp.float32)

    x128 = jnp.zeros((NPAD, 128), jnp.float32).at[:N0, :4].set(x)
    agg1 = _scatter128(src_g, dst_l, x128, z)[:N0, :4]
    h1 = jax.nn.relu(agg1 @ c1_rel_w.T + c1_rel_b + x @ c1_root_w.T)

    s1 = jnp.tanh((h1 @ p1_w) / jnp.linalg.norm(p1_w))
    v1, i1 = lax.top_k(s1.reshape(B, NPG), K1)
    rows = jnp.arange(B)[:, None]
    mask1 = jnp.zeros((B, NPG), bool).at[rows, i1].set(True).reshape(-1)
    hs1 = h1 * s1[:, None]
    g1 = hs1 * mask1[:, None].astype(jnp.float32)
    x1 = jnp.concatenate([
        jnp.where(mask1[:, None], hs1, -jnp.inf).reshape(B, NPG, 128).max(1),
        g1.reshape(B, NPG, 128).sum(1) / K1], axis=1)

    g1p = jnp.zeros((NPAD, 128), jnp.float32).at[:N0].set(g1)
    agg2 = _scatter128(src_g, dst_l, g1p, z)[:N0]
    h2 = jax.nn.relu(agg2 @ c2_rel_w.T + c2_rel_b + g1 @ c2_root_w.T)

    s2 = jnp.tanh((h2 @ p2_w) / jnp.linalg.norm(p2_w))
    s2_ranked = s2.reshape(B, NPG)[rows, i1]
    v2, i2r = lax.top_k(s2_ranked, K2)
    i2 = i1[rows, i2r]
    mask2 = jnp.zeros((B, NPG), bool).at[rows, i2].set(True).reshape(-1)
    hs2 = h2 * s2[:, None]
    g2 = hs2 * mask2[:, None].astype(jnp.float32)
    x2 = jnp.concatenate([
        jnp.where(mask2[:, None], hs2, -jnp.inf).reshape(B, NPG, 128).max(1),
        g2.reshape(B, NPG, 128).sum(1) / K2], axis=1)
    xg = x1 + x2

    xt = lax.conv_general_dilated(target, cv_w, (1,), 'VALID',
                                  dimension_numbers=('NCH', 'OIH', 'NCH'))
    xt = jax.nn.relu(xt + cv_b[None, :, None])
    xt = xt.reshape(B, 128, 19, 5).mean(-1)
    xt = jnp.transpose(xt, (0, 2, 1))
    of = _gru_dir(xt, gru_wih_f, gru_whh_f, gru_bih_f, gru_bhh_f, False)
    orv = _gru_dir(xt, gru_wih_r, gru_whh_r, gru_bih_r, gru_bhh_r, True)
    xt = jnp.concatenate([of, orv], axis=-1).reshape(B, -1)

    xc = jnp.concatenate([xg, xt], axis=1)
    xc = jax.nn.relu(xc @ d1_w.T + d1_b)
    logits = xc @ d3_w.T + d3_b
    return jax.nn.log_softmax(logits, axis=-1)


# speedup vs baseline: 1.3761x; 1.0048x over previous
import functools

import jax
import jax.numpy as jnp
from jax import lax
from jax.experimental import pallas as pl
from jax.experimental.pallas import tpu as pltpu
from jax.experimental.pallas import tpu_sc as plsc

B = 50; N0 = 10000; NPG = 200; K1 = 160; K2 = 128; E = 160000

NPAD = 10240
NOWN = 32
OWNR = NPAD // NOWN
HALF = NPAD // 2
CAP = 6144
CL = 128
NCHUNK = CAP // CL
PADROW = HALF


def _make_edge_scatter(F):
    mesh = plsc.VectorSubcoreMesh(core_axis_name="c", subcore_axis_name="s")

    NBUF = 4

    @functools.partial(
        pl.kernel, mesh=mesh,
        out_type=jax.ShapeDtypeStruct((NPAD, F), jnp.float32),
        scratch_types=[
            pltpu.VMEM((NCHUNK, CL), jnp.int32),
            pltpu.VMEM((NCHUNK, CL), jnp.int32),
            pltpu.VMEM((NBUF, CL, F), jnp.float32),
            pltpu.VMEM_SHARED((HALF + 8, F), jnp.float32),
        ] + [pltpu.SemaphoreType.DMA] * NBUF,
    )
    def k(src_hbm, dst_hbm, rows_hbm, zeros_hbm, out_hbm,
          src_v, dst_v, rows_v, agg_sh, *sems):
        c = lax.axis_index("c")
        s = lax.axis_index("s")
        base = s * OWNR
        pltpu.sync_copy(src_hbm.at[c, s], src_v)
        pltpu.sync_copy(dst_hbm.at[c, s], dst_v)
        for z in range(OWNR // 64):
            pltpu.sync_copy(zeros_hbm, agg_sh.at[pl.ds(base + z * 64, 64)])
        plsc.subcore_barrier()

        def gather(j, b):
            return pltpu.make_async_copy(
                rows_hbm.at[src_v.at[j]], rows_v.at[b], sems[b])

        for b in range(NBUF):
            gather(b, b).start()

        def outer(g, carry):
            j0 = g * NBUF
            for b in range(NBUF):
                j = j0 + b
                gather(j, b).wait()
                pltpu.sync_copy(rows_v.at[b], agg_sh.at[dst_v.at[j]], add=True)
                gather(lax.rem(j + NBUF, NCHUNK), b).start()
            return carry

        lax.fori_loop(0, NCHUNK // NBUF, outer, 0)
        for b in range(NBUF):
            gather(b, b).wait()
        plsc.subcore_barrier()
        pltpu.sync_copy(agg_sh.at[pl.ds(base, OWNR)],
                        out_hbm.at[pl.ds(c * HALF + base, OWNR)])

    return k


_scatter128 = _make_edge_scatter(128)


def _bin_edges(src, dst):
    src = src.astype(jnp.int32)
    dst = dst.astype(jnp.int32)
    owner = dst // OWNR
    order = jnp.argsort(owner, stable=True).astype(jnp.int32)
    src_s = src[order]
    dst_s = dst[order]
    owner_s = owner[order]
    counts = jnp.bincount(owner, length=NOWN)
    starts = jnp.concatenate([jnp.zeros((1,), counts.dtype),
                              jnp.cumsum(counts)[:-1]])
    slot = jnp.arange(E, dtype=jnp.int32) - starts[owner_s].astype(jnp.int32)
    flat = owner_s * CAP + slot
    src_p = jnp.full((NOWN * CAP,), N0, jnp.int32).at[flat].set(
        src_s, unique_indices=True)
    dst_l = dst_s - (owner_s // 16) * HALF
    dst_p = jnp.full((NOWN * CAP,), PADROW, jnp.int32).at[flat].set(
        dst_l, unique_indices=True)
    shape = (2, 16, NCHUNK, CL)
    return src_p.reshape(shape), dst_p.reshape(shape)


def _tail_body(target_ref, xg_ref, cv_w_ref, cv_b_ref,
               wih_f_ref, whh_f_ref, bih_f_ref, bhh_f_ref,
               wih_r_ref, whh_r_ref, bih_r_ref, bhh_r_ref,
               d1_w_ref, d1_b_ref, d3_w_ref, d3_b_ref, out_ref):
    t = target_ref[...]
    conv = jnp.zeros((B, 95, 128), jnp.float32)
    cvw = cv_w_ref[...]
    for k in range(3):
        conv += lax.dot_general(t[:, :, k:k + 95], cvw[:, :, k],
                                (((1,), (1,)), ((), ())))
    conv = jax.nn.relu(conv + cv_b_ref[...][None, None, :])
    xt = conv.reshape(B, 19, 5, 128).mean(2)

    def gru(wih_ref, whh_ref, bih_ref, bhh_ref, reverse):
        gi = lax.dot_general(xt, wih_ref[...],
                             (((2,), (1,)), ((), ()))) + bih_ref[...]
        whh = whh_ref[...]
        bhh = bhh_ref[...]
        h = jnp.zeros((B, 120), jnp.float32)
        hs = [None] * 19
        order = range(18, -1, -1) if reverse else range(19)
        for step in order:
            g = gi[:, step]
            gh = lax.dot_general(h, whh, (((1,), (1,)), ((), ()))) + bhh
            r = jax.nn.sigmoid(g[:, 0:120] + gh[:, 0:120])
            z = jax.nn.sigmoid(g[:, 120:240] + gh[:, 120:240])
            n = jnp.tanh(g[:, 240:360] + r * gh[:, 240:360])
            h = (1 - z) * n + z * h
            hs[step] = h
        return hs

    hf = gru(wih_f_ref, whh_f_ref, bih_f_ref, bhh_f_ref, False)
    hr = gru(wih_r_ref, whh_r_ref, bih_r_ref, bhh_r_ref, True)
    seq = jnp.concatenate([jnp.concatenate([hf[s], hr[s]], axis=1)
                           for s in range(19)], axis=1)
    xc = jnp.concatenate([xg_ref[...], seq], axis=1)
    xc = jax.nn.relu(lax.dot_general(xc, d1_w_ref[...],
                                     (((1,), (1,)), ((), ()))) + d1_b_ref[...])
    logits = lax.dot_general(xc, d3_w_ref[...],
                             (((1,), (1,)), ((), ()))) + d3_b_ref[...]
    m = jnp.max(logits, axis=1, keepdims=True)
    e = logits - m
    out_ref[...] = e - jnp.log(jnp.sum(jnp.exp(e), axis=1, keepdims=True))


def _tail(target, xg, cv_w, cv_b, wf, whf, bf, bhf, wr, whr, br, bhr,
          d1_w, d1_b, d3_w, d3_b):
    return pl.pallas_call(
        _tail_body,
        out_shape=jax.ShapeDtypeStruct((B, 2), jnp.float32),
    )(target, xg, cv_w, cv_b, wf, whf, bf, bhf, wr, whr, br, bhr,
      d1_w, d1_b, d3_w, d3_b)


def kernel(x, edge_index, batch, target, c1_rel_w, c1_rel_b, c1_root_w, p1_w,
           c2_rel_w, c2_rel_b, c2_root_w, p2_w, cv_w, cv_b,
           gru_wih_f, gru_whh_f, gru_bih_f, gru_bhh_f,
           gru_wih_r, gru_whh_r, gru_bih_r, gru_bhh_r, d1_w, d1_b, d3_w, d3_b):
    src, dst = edge_index[0], edge_index[1]
    src_g, dst_l = _bin_edges(src, dst)
    z = jnp.zeros((64, 128), jnp.float32)

    x128 = jnp.zeros((NPAD, 128), jnp.float32).at[:N0, :4].set(x)
    agg1 = _scatter128(src_g, dst_l, x128, z)[:N0, :4]
    h1 = jax.nn.relu(agg1 @ c1_rel_w.T + c1_rel_b + x @ c1_root_w.T)

    s1 = jnp.tanh((h1 @ p1_w) / jnp.linalg.norm(p1_w))
    v1, i1 = lax.top_k(s1.reshape(B, NPG), K1)
    rows = jnp.arange(B)[:, None]
    mask1 = jnp.zeros((B, NPG), bool).at[rows, i1].set(True).reshape(-1)
    hs1 = h1 * s1[:, None]
    g1 = hs1 * mask1[:, None].astype(jnp.float32)
    x1 = jnp.concatenate([
        jnp.where(mask1[:, None], hs1, -jnp.inf).reshape(B, NPG, 128).max(1),
        g1.reshape(B, NPG, 128).sum(1) / K1], axis=1)

    g1p = jnp.zeros((NPAD, 128), jnp.float32).at[:N0].set(g1)
    agg2 = _scatter128(src_g, dst_l, g1p, z)[:N0]
    h2 = jax.nn.relu(agg2 @ c2_rel_w.T + c2_rel_b + g1 @ c2_root_w.T)

    s2 = jnp.tanh((h2 @ p2_w) / jnp.linalg.norm(p2_w))
    s2_ranked = s2.reshape(B, NPG)[rows, i1]
    v2, i2r = lax.top_k(s2_ranked, K2)
    i2 = i1[rows, i2r]
    mask2 = jnp.zeros((B, NPG), bool).at[rows, i2].set(True).reshape(-1)
    hs2 = h2 * s2[:, None]
    g2 = hs2 * mask2[:, None].astype(jnp.float32)
    x2 = jnp.concatenate([
        jnp.where(mask2[:, None], hs2, -jnp.inf).reshape(B, NPG, 128).max(1),
        g2.reshape(B, NPG, 128).sum(1) / K2], axis=1)
    xg = x1 + x2

    return _tail(target, xg, cv_w, cv_b,
                 gru_wih_f, gru_whh_f, gru_bih_f, gru_bhh_f,
                 gru_wih_r, gru_whh_r, gru_bih_r, gru_bhh_r,
                 d1_w, d1_b, d3_w, d3_b)
